# Initial kernel scaffold; baseline (speedup 1.0000x reference)
import jax
import jax.numpy as jnp
from jax.experimental import pallas as pl


def kernel(x, prev_h, norm, W, att_w, att_b, loop_w, evolve_loop_w, edge_index, etype):
    raise NotImplementedError("write your pallas kernel here")


# speedup vs baseline: 3.7485x; 3.7485x over previous
import jax
import jax.numpy as jnp
from jax import lax
from jax.experimental import pallas as pl
from jax.experimental.pallas import tpu as pltpu
from jax.experimental.pallas import tpu_sc as plsc

N = 10000
E = 320000
D = 128
NC = 2
NS = 16
NW = NC * NS
EP = E // NW
C = 80
NCHUNK = EP // C
NP = 10240
RPT = NP // NS
NB = 1000
NBE = 1024


def _att_body(x_ref, v_ref, b_ref, o_ref):
    o_ref[...] = jnp.dot(x_ref[...], v_ref[...],
                         preferred_element_type=jnp.float32) + b_ref[...]


def _att_scalars(x, vmat, bvec):
    return pl.pallas_call(
        _att_body,
        grid=(N // NB,),
        in_specs=[
            pl.BlockSpec((NB, D), lambda i: (i, 0)),
            pl.BlockSpec((D, D), lambda i: (0, 0)),
            pl.BlockSpec((1, D), lambda i: (0, 0)),
        ],
        out_specs=pl.BlockSpec((NB, D), lambda i: (i, 0)),
        out_shape=jax.ShapeDtypeStruct((N, D), jnp.float32),
    )(x, vmat, bvec)


def _sc_body(x_hbm, asrc_hbm, adst_hbm, src_hbm, dst_hbm, sp_hbm,
             acc_out, flg_out,
             asrc_v, adst_v, srcb, dstb, spb, attb, rows, flagb, sem,
             accS):
    cid = lax.axis_index("c")
    sid = lax.axis_index("s")
    wid = sid * NC + cid
    zero16 = jnp.zeros((16,), jnp.float32)
    one16 = jnp.full((16,), 1.0, jnp.float32)

    pltpu.sync_copy(asrc_hbm, asrc_v)
    pltpu.sync_copy(adst_hbm, adst_v)

    @pl.loop(0, C)
    def _zrows(e):
        for f in range(D // 16):
            rows[e, pl.ds(f * 16, 16)] = zero16

    @pl.loop(0, NP // 16)
    def _zflag(g):
        flagb[pl.ds(g * 16, 16)] = zero16

    base = sid * RPT
    for k in range(RPT // C):
        pltpu.sync_copy(rows, accS.at[pl.ds(base + k * C, C)])

    plsc.subcore_barrier()

    ebase = wid * EP

    @pl.loop(0, NCHUNK)
    def _chunk(i):
        off = ebase + i * C
        pltpu.sync_copy(src_hbm.at[pl.ds(off, C)], srcb)
        pltpu.sync_copy(dst_hbm.at[pl.ds(off, C)], dstb)
        pltpu.sync_copy(sp_hbm.at[pl.ds(off, C)], spb)
        pltpu.async_copy(x_hbm.at[spb], rows, sem).wait()

        for g in range(C // 16):
            sv = srcb[pl.ds(g * 16, 16)]
            dv = dstb[pl.ds(g * 16, 16)]
            z = plsc.load_gather(asrc_v, [sv]) + plsc.load_gather(adst_v, [dv])
            z = jnp.maximum(z, jnp.float32(0))
            attb[pl.ds(g * 16, 16)] = 1.0 / (1.0 + jnp.exp(-z))
            plsc.store_scatter(flagb, [dv], one16)

        @pl.loop(0, C)
        def _scale(e):
            av = plsc.load_gather(attb, [jnp.full((16,), e, jnp.int32)])
            for f in range(D // 16):
                rows[e, pl.ds(f * 16, 16)] = rows[e, pl.ds(f * 16, 16)] * av

        pltpu.sync_copy(rows, accS.at[dstb], add=True)

    plsc.subcore_barrier()

    for k in range(RPT // C):
        r = base + k * C
        pltpu.sync_copy(accS.at[pl.ds(r, C)], rows)
        pltpu.sync_copy(rows, acc_out.at[cid, pl.ds(r, C)])
    pltpu.sync_copy(flagb, flg_out.at[cid, sid])


_sc_edges = pl.kernel(
    _sc_body,
    out_type=(
        jax.ShapeDtypeStruct((NC, NP, D), jnp.float32),
        jax.ShapeDtypeStruct((NC, NS, NP), jnp.float32),
    ),
    mesh=plsc.VectorSubcoreMesh(core_axis_name="c", subcore_axis_name="s"),
    compiler_params=pltpu.CompilerParams(needs_layout_passes=False),
    scratch_types=[
        pltpu.VMEM((N,), jnp.float32),
        pltpu.VMEM((N,), jnp.float32),
        pltpu.VMEM((C,), jnp.int32),
        pltpu.VMEM((C,), jnp.int32),
        pltpu.VMEM((C,), jnp.int32),
        pltpu.VMEM((C,), jnp.float32),
        pltpu.VMEM((C, D), jnp.float32),
        pltpu.VMEM((NP,), jnp.float32),
        pltpu.SemaphoreType.DMA,
        pltpu.VMEM_SHARED((NP, D), jnp.float32),
    ],
)


def _out_body(acc_ref, flg_ref, x_ref, norm_ref, w_ref, lw_ref, ew_ref, o_ref):
    acc = acc_ref[0] + acc_ref[1]
    flg = jnp.sum(flg_ref[...], axis=(0, 1))[:, None]
    agg = jnp.dot(acc, w_ref[...], preferred_element_type=jnp.float32)
    x = x_ref[...]
    lm = jnp.where(flg > 0,
                   jnp.dot(x, lw_ref[...], preferred_element_type=jnp.float32),
                   jnp.dot(x, ew_ref[...], preferred_element_type=jnp.float32))
    o_ref[...] = jnp.tanh(agg * norm_ref[...] + lm)


def _epilogue(acc2, flg2, x, norm2, W, loop_w, evolve_loop_w):
    return pl.pallas_call(
        _out_body,
        grid=(NP // NBE,),
        in_specs=[
            pl.BlockSpec((NC, NBE, D), lambda i: (0, i, 0)),
            pl.BlockSpec((NC, NS, NBE), lambda i: (0, 0, i)),
            pl.BlockSpec((NBE, D), lambda i: (i, 0)),
            pl.BlockSpec((NBE, 1), lambda i: (i, 0)),
            pl.BlockSpec((D, D), lambda i: (0, 0)),
            pl.BlockSpec((D, D), lambda i: (0, 0)),
            pl.BlockSpec((D, D), lambda i: (0, 0)),
        ],
        out_specs=pl.BlockSpec((NBE, D), lambda i: (i, 0)),
        out_shape=jax.ShapeDtypeStruct((NP, D), jnp.float32),
    )(acc2, flg2, x, norm2, W, loop_w, evolve_loop_w)


@jax.jit
def kernel(x, prev_h, norm, W, att_w, att_b, loop_w, evolve_loop_w,
           edge_index, etype):
    src = edge_index[0]
    dst = edge_index[1]
    sp = src[jnp.argsort(etype)].astype(jnp.int32)

    vmat = jnp.zeros((D, D), jnp.float32)
    vmat = vmat.at[:, 0].set(att_w[0, :D]).at[:, 1].set(att_w[0, D:])
    bvec = jnp.zeros((1, D), jnp.float32).at[0, 1].set(att_b[0])

    a = _att_scalars(x, vmat, bvec)
    a_src = a[:, 0]
    a_dst = a[:, 1]

    acc2, flg2 = _sc_edges(x, a_src, a_dst,
                           src.astype(jnp.int32), dst.astype(jnp.int32), sp)

    x_pad = jnp.pad(x, ((0, NP - N), (0, 0)))
    norm_pad = jnp.pad(norm, (0, NP - N)).reshape(NP, 1)
    out = _epilogue(acc2, flg2, x_pad, norm_pad, W, loop_w, evolve_loop_w)
    return out[:N]


# speedup vs baseline: 3.9586x; 1.0561x over previous
import jax
import jax.numpy as jnp
from jax import lax
from jax.experimental import pallas as pl
from jax.experimental.pallas import tpu as pltpu
from jax.experimental.pallas import tpu_sc as plsc

N = 10000
E = 320000
D = 128
NC = 2
NS = 16
NW = NC * NS
EP = E // NW
C = 128
NCHUNK = EP // C
CR = EP - NCHUNK * C
NP = 10112
RPT = NP // NS
NB = 1000
NBE = 128


def _att_body(x_ref, v_ref, b_ref, o_ref):
    o_ref[...] = jnp.dot(x_ref[...], v_ref[...],
                         preferred_element_type=jnp.float32) + b_ref[...]


def _att_scalars(x, vmat, bvec):
    return pl.pallas_call(
        _att_body,
        grid=(N // NB,),
        in_specs=[
            pl.BlockSpec((NB, D), lambda i: (i, 0)),
            pl.BlockSpec((D, D), lambda i: (0, 0)),
            pl.BlockSpec((1, D), lambda i: (0, 0)),
        ],
        out_specs=pl.BlockSpec((NB, D), lambda i: (i, 0)),
        out_shape=jax.ShapeDtypeStruct((N, D), jnp.float32),
    )(x, vmat, bvec)


def _sc_body(x_hbm, asrc_hbm, adst_hbm, src_hbm, dst_hbm, sp_hbm,
             acc_out, flg_out,
             asrc_v, adst_v, srcb, dstb, spb, attb, rows,
             srcb_r, dstb_r, spb_r, rows_r, flagb, sem,
             accS):
    cid = lax.axis_index("c")
    sid = lax.axis_index("s")
    wid = sid * NC + cid
    zero16 = jnp.zeros((16,), jnp.float32)
    one16 = jnp.full((16,), 1.0, jnp.float32)

    pltpu.sync_copy(asrc_hbm, asrc_v)
    pltpu.sync_copy(adst_hbm, adst_v)

    @pl.loop(0, C)
    def _zrows(e):
        for f in range(D // 16):
            rows[e, pl.ds(f * 16, 16)] = zero16

    @pl.loop(0, NP // 16)
    def _zflag(g):
        flagb[pl.ds(g * 16, 16)] = zero16

    base = sid * RPT
    for k in range(RPT // C):
        pltpu.sync_copy(rows, accS.at[pl.ds(base + k * C, C)])
    ZR = RPT - (RPT // C) * C
    pltpu.sync_copy(rows.at[pl.ds(0, ZR)],
                    accS.at[pl.ds(base + RPT - ZR, ZR)])

    plsc.subcore_barrier()

    ebase = wid * EP

    @pl.loop(0, NCHUNK)
    def _chunk(i):
        off = ebase + i * C
        pltpu.sync_copy(src_hbm.at[pl.ds(off, C)], srcb)
        pltpu.sync_copy(dst_hbm.at[pl.ds(off, C)], dstb)
        pltpu.sync_copy(sp_hbm.at[pl.ds(off, C)], spb)
        pltpu.async_copy(x_hbm.at[spb], rows, sem).wait()

        for g in range(C // 16):
            sv = srcb[pl.ds(g * 16, 16)]
            dv = dstb[pl.ds(g * 16, 16)]
            z = plsc.load_gather(asrc_v, [sv]) + plsc.load_gather(adst_v, [dv])
            z = jnp.maximum(z, jnp.float32(0))
            attb[pl.ds(g * 16, 16)] = 1.0 / (1.0 + jnp.exp(-z))
            plsc.store_scatter(flagb, [dv], one16)

        @pl.loop(0, C)
        def _scale(e):
            av = plsc.load_gather(attb, [jnp.full((16,), e, jnp.int32)])
            for f in range(D // 16):
                rows[e, pl.ds(f * 16, 16)] = rows[e, pl.ds(f * 16, 16)] * av

        pltpu.sync_copy(rows, accS.at[dstb], add=True)

    roff = ebase + NCHUNK * C
    pltpu.sync_copy(src_hbm.at[pl.ds(roff, CR)], srcb_r)
    pltpu.sync_copy(dst_hbm.at[pl.ds(roff, CR)], dstb_r)
    pltpu.sync_copy(sp_hbm.at[pl.ds(roff, CR)], spb_r)
    pltpu.async_copy(x_hbm.at[spb_r], rows_r, sem).wait()
    sv = srcb_r[pl.ds(0, 16)]
    dv = dstb_r[pl.ds(0, 16)]
    z = plsc.load_gather(asrc_v, [sv]) + plsc.load_gather(adst_v, [dv])
    z = jnp.maximum(z, jnp.float32(0))
    attb[pl.ds(0, 16)] = 1.0 / (1.0 + jnp.exp(-z))
    plsc.store_scatter(flagb, [dv], one16)

    @pl.loop(0, CR)
    def _scale_r(e):
        av = plsc.load_gather(attb, [jnp.full((16,), e, jnp.int32)])
        for f in range(D // 16):
            rows_r[e, pl.ds(f * 16, 16)] = rows_r[e, pl.ds(f * 16, 16)] * av

    pltpu.sync_copy(rows_r, accS.at[dstb_r], add=True)

    plsc.subcore_barrier()

    for k in range(RPT // C):
        r = base + k * C
        pltpu.sync_copy(accS.at[pl.ds(r, C)], rows)
        pltpu.sync_copy(rows, acc_out.at[cid, pl.ds(r, C)])
    TR = RPT - (RPT // C) * C
    pltpu.sync_copy(accS.at[pl.ds(base + RPT - TR, TR)], rows.at[pl.ds(0, TR)])
    pltpu.sync_copy(rows.at[pl.ds(0, TR)],
                    acc_out.at[cid, pl.ds(base + RPT - TR, TR)])
    pltpu.sync_copy(flagb, flg_out.at[cid, sid])


_sc_edges = pl.kernel(
    _sc_body,
    out_type=(
        jax.ShapeDtypeStruct((NC, NP, D), jnp.float32),
        jax.ShapeDtypeStruct((NC, NS, NP), jnp.float32),
    ),
    mesh=plsc.VectorSubcoreMesh(core_axis_name="c", subcore_axis_name="s"),
    compiler_params=pltpu.CompilerParams(needs_layout_passes=False),
    scratch_types=[
        pltpu.VMEM((N,), jnp.float32),
        pltpu.VMEM((N,), jnp.float32),
        pltpu.VMEM((C,), jnp.int32),
        pltpu.VMEM((C,), jnp.int32),
        pltpu.VMEM((C,), jnp.int32),
        pltpu.VMEM((C,), jnp.float32),
        pltpu.VMEM((C, D), jnp.float32),
        pltpu.VMEM((16,), jnp.int32),
        pltpu.VMEM((16,), jnp.int32),
        pltpu.VMEM((16,), jnp.int32),
        pltpu.VMEM((16, D), jnp.float32),
        pltpu.VMEM((NP,), jnp.float32),
        pltpu.SemaphoreType.DMA,
        pltpu.VMEM_SHARED((NP, D), jnp.float32),
    ],
)


def _out_body(acc_ref, flg_ref, x_ref, norm_ref, w_ref, lw_ref, ew_ref, o_ref):
    acc = acc_ref[0] + acc_ref[1]
    flg = jnp.sum(flg_ref[...], axis=(0, 1))[:, None]
    agg = jnp.dot(acc, w_ref[...], preferred_element_type=jnp.float32)
    x = x_ref[...]
    lm = jnp.where(flg > 0,
                   jnp.dot(x, lw_ref[...], preferred_element_type=jnp.float32),
                   jnp.dot(x, ew_ref[...], preferred_element_type=jnp.float32))
    o_ref[...] = jnp.tanh(agg * norm_ref[...] + lm)


def _epilogue(acc2, flg2, x, norm2, W, loop_w, evolve_loop_w):
    return pl.pallas_call(
        _out_body,
        grid=(NP // NBE,),
        in_specs=[
            pl.BlockSpec((NC, NBE, D), lambda i: (0, i, 0)),
            pl.BlockSpec((NC, NS, NBE), lambda i: (0, 0, i)),
            pl.BlockSpec((NBE, D), lambda i: (i, 0)),
            pl.BlockSpec((NBE, 1), lambda i: (i, 0)),
            pl.BlockSpec((D, D), lambda i: (0, 0)),
            pl.BlockSpec((D, D), lambda i: (0, 0)),
            pl.BlockSpec((D, D), lambda i: (0, 0)),
        ],
        out_specs=pl.BlockSpec((NBE, D), lambda i: (i, 0)),
        out_shape=jax.ShapeDtypeStruct((NP, D), jnp.float32),
    )(acc2, flg2, x, norm2, W, loop_w, evolve_loop_w)


@jax.jit
def kernel(x, prev_h, norm, W, att_w, att_b, loop_w, evolve_loop_w,
           edge_index, etype):
    src = edge_index[0]
    dst = edge_index[1]
    sp = src[jnp.argsort(etype)].astype(jnp.int32)

    vmat = jnp.zeros((D, D), jnp.float32)
    vmat = vmat.at[:, 0].set(att_w[0, :D]).at[:, 1].set(att_w[0, D:])
    bvec = jnp.zeros((1, D), jnp.float32).at[0, 1].set(att_b[0])

    a = _att_scalars(x, vmat, bvec)
    a_src = a[:, 0]
    a_dst = a[:, 1]

    acc2, flg2 = _sc_edges(x, a_src, a_dst,
                           src.astype(jnp.int32), dst.astype(jnp.int32), sp)

    x_pad = jnp.pad(x, ((0, NP - N), (0, 0)))
    norm_pad = jnp.pad(norm, (0, NP - N)).reshape(NP, 1)
    out = _epilogue(acc2, flg2, x_pad, norm_pad, W, loop_w, evolve_loop_w)
    return out[:N]


# speedup vs baseline: 4.0092x; 1.0128x over previous
import jax
import jax.numpy as jnp
from jax import lax
from jax.experimental import pallas as pl
from jax.experimental.pallas import tpu as pltpu
from jax.experimental.pallas import tpu_sc as plsc

N = 10000
E = 320000
D = 128
NC = 2
NS = 16
NW = NC * NS
EP = E // NW
C = 128
NCHUNK = EP // C
CR = EP - NCHUNK * C
NP = 10112
RPT = NP // NS
NB = 1000
NBE = 128


def _att_body(x_ref, v_ref, b_ref, o_ref):
    o_ref[...] = jnp.dot(x_ref[...], v_ref[...],
                         preferred_element_type=jnp.float32) + b_ref[...]


def _att_scalars(x, vmat, bvec):
    return pl.pallas_call(
        _att_body,
        grid=(N // NB,),
        in_specs=[
            pl.BlockSpec((NB, D), lambda i: (i, 0)),
            pl.BlockSpec((D, D), lambda i: (0, 0)),
            pl.BlockSpec((1, D), lambda i: (0, 0)),
        ],
        out_specs=pl.BlockSpec((NB, D), lambda i: (i, 0)),
        out_shape=jax.ShapeDtypeStruct((N, D), jnp.float32),
    )(x, vmat, bvec)


def _sc_body(x_hbm, asrc_hbm, adst_hbm, src_hbm, dst_hbm, sp_hbm,
             acc_out, flg_out,
             asrc_v, adst_v, srcb, dstb, spb, attb, rows,
             srcb_r, dstb_r, spb_r, rows_r, flagb, sem,
             accS):
    cid = lax.axis_index("c")
    sid = lax.axis_index("s")
    wid = sid * NC + cid
    zero16 = jnp.zeros((16,), jnp.float32)
    one16 = jnp.full((16,), 1.0, jnp.float32)

    pltpu.sync_copy(asrc_hbm, asrc_v)
    pltpu.sync_copy(adst_hbm, adst_v)

    @pl.loop(0, C)
    def _zrows(e):
        for f in range(D // 16):
            rows[e, pl.ds(f * 16, 16)] = zero16

    @pl.loop(0, NP // 16)
    def _zflag(g):
        flagb[pl.ds(g * 16, 16)] = zero16

    base = sid * RPT
    for k in range(RPT // C):
        pltpu.sync_copy(rows, accS.at[pl.ds(base + k * C, C)])
    ZR = RPT - (RPT // C) * C
    pltpu.sync_copy(rows.at[pl.ds(0, ZR)],
                    accS.at[pl.ds(base + RPT - ZR, ZR)])

    plsc.subcore_barrier()

    ebase = wid * EP

    @pl.loop(0, NCHUNK)
    def _chunk(i):
        off = ebase + i * C
        pltpu.sync_copy(src_hbm.at[pl.ds(off, C)], srcb)
        pltpu.sync_copy(dst_hbm.at[pl.ds(off, C)], dstb)
        pltpu.sync_copy(sp_hbm.at[pl.ds(off, C)], spb)
        pltpu.async_copy(x_hbm.at[spb], rows, sem).wait()

        for g in range(C // 16):
            sv = srcb[pl.ds(g * 16, 16)]
            dv = dstb[pl.ds(g * 16, 16)]
            z = plsc.load_gather(asrc_v, [sv]) + plsc.load_gather(adst_v, [dv])
            z = jnp.maximum(z, jnp.float32(0))
            attb[pl.ds(g * 16, 16)] = 1.0 / (1.0 + jnp.exp(-z))
            plsc.store_scatter(flagb, [dv], one16)

        @pl.loop(0, C, unroll=4)
        def _scale(e):
            av = plsc.load_gather(attb, [jnp.full((16,), e, jnp.int32)])
            for f in range(D // 16):
                rows[e, pl.ds(f * 16, 16)] = rows[e, pl.ds(f * 16, 16)] * av

        pltpu.sync_copy(rows, accS.at[dstb], add=True)

    roff = ebase + NCHUNK * C
    pltpu.sync_copy(src_hbm.at[pl.ds(roff, CR)], srcb_r)
    pltpu.sync_copy(dst_hbm.at[pl.ds(roff, CR)], dstb_r)
    pltpu.sync_copy(sp_hbm.at[pl.ds(roff, CR)], spb_r)
    pltpu.async_copy(x_hbm.at[spb_r], rows_r, sem).wait()
    sv = srcb_r[pl.ds(0, 16)]
    dv = dstb_r[pl.ds(0, 16)]
    z = plsc.load_gather(asrc_v, [sv]) + plsc.load_gather(adst_v, [dv])
    z = jnp.maximum(z, jnp.float32(0))
    attb[pl.ds(0, 16)] = 1.0 / (1.0 + jnp.exp(-z))
    plsc.store_scatter(flagb, [dv], one16)

    @pl.loop(0, CR)
    def _scale_r(e):
        av = plsc.load_gather(attb, [jnp.full((16,), e, jnp.int32)])
        for f in range(D // 16):
            rows_r[e, pl.ds(f * 16, 16)] = rows_r[e, pl.ds(f * 16, 16)] * av

    pltpu.sync_copy(rows_r, accS.at[dstb_r], add=True)

    plsc.subcore_barrier()

    for k in range(RPT // C):
        r = base + k * C
        pltpu.sync_copy(accS.at[pl.ds(r, C)], rows)
        pltpu.sync_copy(rows, acc_out.at[cid, pl.ds(r, C)])
    TR = RPT - (RPT // C) * C
    pltpu.sync_copy(accS.at[pl.ds(base + RPT - TR, TR)], rows.at[pl.ds(0, TR)])
    pltpu.sync_copy(rows.at[pl.ds(0, TR)],
                    acc_out.at[cid, pl.ds(base + RPT - TR, TR)])
    pltpu.sync_copy(flagb, flg_out.at[cid, sid])


_sc_edges = pl.kernel(
    _sc_body,
    out_type=(
        jax.ShapeDtypeStruct((NC, NP, D), jnp.float32),
        jax.ShapeDtypeStruct((NC, NS, NP), jnp.float32),
    ),
    mesh=plsc.VectorSubcoreMesh(core_axis_name="c", subcore_axis_name="s"),
    compiler_params=pltpu.CompilerParams(needs_layout_passes=False),
    scratch_types=[
        pltpu.VMEM((N,), jnp.float32),
        pltpu.VMEM((N,), jnp.float32),
        pltpu.VMEM((C,), jnp.int32),
        pltpu.VMEM((C,), jnp.int32),
        pltpu.VMEM((C,), jnp.int32),
        pltpu.VMEM((C,), jnp.float32),
        pltpu.VMEM((C, D), jnp.float32),
        pltpu.VMEM((16,), jnp.int32),
        pltpu.VMEM((16,), jnp.int32),
        pltpu.VMEM((16,), jnp.int32),
        pltpu.VMEM((16, D), jnp.float32),
        pltpu.VMEM((NP,), jnp.float32),
        pltpu.SemaphoreType.DMA,
        pltpu.VMEM_SHARED((NP, D), jnp.float32),
    ],
)


def _out_body(acc_ref, flg_ref, x_ref, norm_ref, w_ref, lw_ref, ew_ref, o_ref):
    acc = acc_ref[0] + acc_ref[1]
    flg = jnp.sum(flg_ref[...], axis=(0, 1))[:, None]
    agg = jnp.dot(acc, w_ref[...], preferred_element_type=jnp.float32)
    x = x_ref[...]
    lm = jnp.where(flg > 0,
                   jnp.dot(x, lw_ref[...], preferred_element_type=jnp.float32),
                   jnp.dot(x, ew_ref[...], preferred_element_type=jnp.float32))
    o_ref[...] = jnp.tanh(agg * norm_ref[...] + lm)


def _epilogue(acc2, flg2, x, norm2, W, loop_w, evolve_loop_w):
    return pl.pallas_call(
        _out_body,
        grid=(NP // NBE,),
        in_specs=[
            pl.BlockSpec((NC, NBE, D), lambda i: (0, i, 0)),
            pl.BlockSpec((NC, NS, NBE), lambda i: (0, 0, i)),
            pl.BlockSpec((NBE, D), lambda i: (i, 0)),
            pl.BlockSpec((NBE, 1), lambda i: (i, 0)),
            pl.BlockSpec((D, D), lambda i: (0, 0)),
            pl.BlockSpec((D, D), lambda i: (0, 0)),
            pl.BlockSpec((D, D), lambda i: (0, 0)),
        ],
        out_specs=pl.BlockSpec((NBE, D), lambda i: (i, 0)),
        out_shape=jax.ShapeDtypeStruct((NP, D), jnp.float32),
    )(acc2, flg2, x, norm2, W, loop_w, evolve_loop_w)


@jax.jit
def kernel(x, prev_h, norm, W, att_w, att_b, loop_w, evolve_loop_w,
           edge_index, etype):
    src = edge_index[0]
    dst = edge_index[1]
    sp = src[jnp.argsort(etype)].astype(jnp.int32)

    vmat = jnp.zeros((D, D), jnp.float32)
    vmat = vmat.at[:, 0].set(att_w[0, :D]).at[:, 1].set(att_w[0, D:])
    bvec = jnp.zeros((1, D), jnp.float32).at[0, 1].set(att_b[0])

    a = _att_scalars(x, vmat, bvec)
    a_src = a[:, 0]
    a_dst = a[:, 1]

    acc2, flg2 = _sc_edges(x, a_src, a_dst,
                           src.astype(jnp.int32), dst.astype(jnp.int32), sp)

    x_pad = jnp.pad(x, ((0, NP - N), (0, 0)))
    norm_pad = jnp.pad(norm, (0, NP - N)).reshape(NP, 1)
    out = _epilogue(acc2, flg2, x_pad, norm_pad, W, loop_w, evolve_loop_w)
    return out[:N]
